# baseline (speedup 1.0000x reference)
import functools

import jax
import jax.numpy as jnp
from jax import lax
from jax.experimental import pallas as pl
from jax.experimental.pallas import tpu as pltpu

N_CELL = 10000
N_NET = 10000
N_PIN = 320000
N_NN = 100000
L = 3

_LOG2 = 0.6931471805599453


def _ssp(x):
    return jax.nn.softplus(x) - _LOG2


def _inv_sqrt_deg(idx, n):
    deg = jnp.zeros((n,), jnp.float32).at[idx].add(1.0)
    return jnp.where(deg > 0, deg, 1.0) ** -0.5


def _readout_act_kernel(zd_ref, za_ref, dis_ref, ang_ref):
    zd = zd_ref[...]
    za = za_ref[...]
    dis_ref[...] = jax.nn.softplus(zd)
    ang_ref[...] = 2.0 * jax.nn.sigmoid(za)


def _readout_act(zd, za):
    n = zd.shape[0]
    zd = zd.reshape(n // 1000, 1000)
    za = za.reshape(n // 1000, 1000)
    dis, ang = pl.pallas_call(
        _readout_act_kernel,
        out_shape=[jax.ShapeDtypeStruct(zd.shape, jnp.float32),
                   jax.ShapeDtypeStruct(zd.shape, jnp.float32)],
    )(zd, za)
    return dis.reshape(n), ang.reshape(n)


def kernel(cell_feat, net_feat, pin_feat, pins_edge_index, net_net_edge_index, net_net_pair_matrix, net_cell_pair_matrix, cell_lin_W, cell_lin_b, net_lin_W, net_lin_b, pin_lin_W, pin_lin_b, pins_W, pins_b, fa_W, fa_b, so_W, so_b, cf_node_W, cf_node_b, cf_e1_W, cf_e1_b, cf_e2_W, cf_e2_b, cf_out_W, cf_out_b, net_dis_W, net_dis_b, net_angle_W, net_angle_b, pin_dis_W, pin_dis_b, pin_angle_W, pin_angle_b):
    hc = jnp.tanh(cell_feat @ cell_lin_W + cell_lin_b)
    hn = jnp.tanh(net_feat @ net_lin_W + net_lin_b)
    hp = jnp.tanh(pin_feat @ pin_lin_W + pin_lin_b)

    c_idx, n_idx = pins_edge_index[0], pins_edge_index[1]
    f_src, f_dst = net_net_edge_index[0], net_net_edge_index[1]

    c_cell_out = _inv_sqrt_deg(c_idx, N_CELL)
    c_net_in = _inv_sqrt_deg(n_idx, N_NET)
    c_fsrc = _inv_sqrt_deg(f_src, N_NET)
    c_fdst = _inv_sqrt_deg(f_dst, N_NET)

    for l in range(L):
        h_pins = (hc * c_cell_out[:, None]) @ pins_W[l]
        agg_pins = jnp.zeros((N_NET, h_pins.shape[1]), jnp.float32).at[n_idx].add(h_pins[c_idx])
        net_pins = agg_pins * c_net_in[:, None] + pins_b[l]

        h_fa = (hn * c_fsrc[:, None]) @ fa_W[l]
        agg_fa = jnp.zeros((N_NET, h_fa.shape[1]), jnp.float32).at[f_dst].add(h_fa[f_src])
        net_fa = agg_fa * c_fdst[:, None] + fa_b[l]

        h_so = (hn * c_fdst[:, None]) @ so_W[l]
        agg_so = jnp.zeros((N_NET, h_so.shape[1]), jnp.float32).at[f_src].add(h_so[f_dst])
        net_so = agg_so * c_fsrc[:, None] + so_b[l]

        new_net = jnp.maximum(jnp.maximum(net_pins, net_fa), net_so)

        hv = hn @ cf_node_W[l] + cf_node_b[l]
        he = _ssp(_ssp(hp @ cf_e1_W[l] + cf_e1_b[l]) @ cf_e2_W[l] + cf_e2_b[l])
        h = jnp.zeros((N_CELL, hv.shape[1]), jnp.float32).at[c_idx].add(hv[n_idx] * he)
        new_cell = _ssp(h @ cf_out_W[l] + cf_out_b[l])

        hc, hn = new_cell, new_net

    p0, p1 = net_net_pair_matrix[:, 0], net_net_pair_matrix[:, 1]
    q0, q1 = net_cell_pair_matrix[:, 0], net_cell_pair_matrix[:, 1]

    w_nn = jnp.concatenate([net_dis_W[:64], net_dis_W[64:], net_angle_W[:64], net_angle_W[64:]], axis=1)
    s_nn = hn @ w_nn
    z_nd = s_nn[p0, 0] + s_nn[p1, 1] + net_dis_b[0]
    z_na = s_nn[p0, 2] + s_nn[p1, 3] + net_angle_b[0]

    w_np = jnp.concatenate([pin_dis_W[:64], pin_angle_W[:64]], axis=1)
    w_cp = jnp.concatenate([pin_dis_W[80:], pin_angle_W[80:]], axis=1)
    w_pp = jnp.concatenate([pin_dis_W[64:80], pin_angle_W[64:80]], axis=1)
    s_np = hn @ w_np
    s_cp = hc @ w_cp
    s_pp = hp @ w_pp
    z_pd = s_np[q0, 0] + s_pp[:, 0] + s_cp[q1, 0] + pin_dis_b[0]
    z_pa = s_np[q0, 1] + s_pp[:, 1] + s_cp[q1, 1] + pin_angle_b[0]

    net_dis, net_angle = _readout_act(z_nd, z_na)
    pin_dis, pin_angle = _readout_act(z_pd, z_pa)
    return (net_dis, net_angle, pin_dis, pin_angle)


# speedup vs baseline: 9.4586x; 9.4586x over previous
import functools

import jax
import jax.numpy as jnp
from jax import lax
from jax.experimental import pallas as pl
from jax.experimental.pallas import tpu as pltpu
from jax.experimental.pallas import tpu_sc as plsc

N_CELL = 10000
N_NET = 10000
N_PIN = 320000
N_NN = 100000
L = 3

_NC, _NS = 2, 16
_NW = _NC * _NS
_CHUNK = 128
_N_PAD = 10240
_ROWS_PER_TILE = _N_PAD // _NS

_LOG2 = 0.6931471805599453


def _ssp(x):
    return jax.nn.softplus(x) - _LOG2


def _mesh():
    return plsc.VectorSubcoreMesh(core_axis_name="c", subcore_axis_name="s")


_SC_PARAMS = pltpu.CompilerParams(use_tc_tiling_on_sc=False,
                                  needs_layout_passes=False)


def _seg_sum_call(table, src2d, dst2d, he=None):
    C = src2d.shape[2]
    with_he = he is not None

    scratch = [
        pltpu.VMEM((C, _CHUNK), jnp.int32),
        pltpu.VMEM((C, _CHUNK), jnp.int32),
        pltpu.VMEM((_CHUNK, 64), jnp.float32),
        pltpu.VMEM_SHARED((_N_PAD, 64), jnp.float32),
        pltpu.SemaphoreType.DMA,
    ]
    if with_he:
        scratch.insert(3, pltpu.VMEM((_CHUNK, 64), jnp.float32))

    def body(table_hbm, src_hbm, dst_hbm, zeros_hbm, *rest):
        if with_he:
            (he_hbm, out_hbm, src_v, dst_v, rows_v, he_v, acc_sh, sem) = rest
        else:
            (out_hbm, src_v, dst_v, rows_v, acc_sh, sem) = rest
        cid = lax.axis_index("c")
        sid = lax.axis_index("s")
        pltpu.sync_copy(src_hbm.at[cid, sid], src_v)
        pltpu.sync_copy(dst_hbm.at[cid, sid], dst_v)

        pltpu.sync_copy(zeros_hbm, rows_v)
        row0 = sid * _ROWS_PER_TILE

        def zc(r, _):
            pltpu.sync_copy(rows_v, acc_sh.at[pl.ds(row0 + r * _CHUNK, _CHUNK)])
            return 0

        lax.fori_loop(0, _ROWS_PER_TILE // _CHUNK, zc, 0)
        plsc.subcore_barrier()

        ebase = (cid * _NS + sid) * C * _CHUNK

        def step(j, _):
            pltpu.async_copy(table_hbm.at[src_v.at[j]], rows_v, sem).wait()
            if with_he:
                pltpu.sync_copy(he_hbm.at[pl.ds(ebase + j * _CHUNK, _CHUNK)], he_v)

                def mul(i, _):
                    for q in range(4):
                        s = pl.ds(q * 16, 16)
                        rows_v[i, s] = rows_v[i, s] * he_v[i, s]
                    return 0

                lax.fori_loop(0, _CHUNK, mul, 0)
            pltpu.sync_copy(rows_v, acc_sh.at[dst_v.at[j]], add=True)
            return 0

        lax.fori_loop(0, C, step, 0)
        plsc.subcore_barrier()

        def wb(r, _):
            o = row0 + r * _CHUNK
            pltpu.sync_copy(acc_sh.at[pl.ds(o, _CHUNK)], rows_v)
            pltpu.sync_copy(rows_v, out_hbm.at[cid, pl.ds(o, _CHUNK)])
            return 0

        lax.fori_loop(0, _ROWS_PER_TILE // _CHUNK, wb, 0)

    k = pl.kernel(
        body,
        out_type=jax.ShapeDtypeStruct((_NC, _N_PAD, 64), jnp.float32),
        mesh=_mesh(),
        scratch_types=scratch,
        compiler_params=_SC_PARAMS,
    )
    zeros = jnp.zeros((_CHUNK, 64), jnp.float32)
    args = (table, src2d, dst2d, zeros) + ((he,) if with_he else ())
    return k(*args)


def _count_call(dst2d):
    C = dst2d.shape[2]

    def body(dst_hbm, ones_hbm, zeros_hbm, out_hbm, dst_v, ones_v, rows_v, acc_sh):
        cid = lax.axis_index("c")
        sid = lax.axis_index("s")
        pltpu.sync_copy(dst_hbm.at[cid, sid], dst_v)
        pltpu.sync_copy(ones_hbm, ones_v)
        pltpu.sync_copy(zeros_hbm, rows_v)
        row0 = sid * _ROWS_PER_TILE
        zrows = _ROWS_PER_TILE // 5

        def zc(r, _):
            pltpu.sync_copy(rows_v, acc_sh.at[pl.ds(row0 + r * zrows, zrows)])
            return 0

        lax.fori_loop(0, _ROWS_PER_TILE // zrows, zc, 0)
        plsc.subcore_barrier()

        def step(j, _):
            pltpu.sync_copy(ones_v, acc_sh.at[dst_v.at[j]], add=True)
            return 0

        lax.fori_loop(0, C, step, 0)
        plsc.subcore_barrier()

        def wb(r, _):
            o = row0 + r * zrows
            pltpu.sync_copy(acc_sh.at[pl.ds(o, zrows)], rows_v)
            pltpu.sync_copy(rows_v, out_hbm.at[cid, pl.ds(o, zrows)])
            return 0

        lax.fori_loop(0, _ROWS_PER_TILE // zrows, wb, 0)

    k = pl.kernel(
        body,
        out_type=jax.ShapeDtypeStruct((_NC, _N_PAD, 8), jnp.float32),
        mesh=_mesh(),
        scratch_types=[
            pltpu.VMEM((C, _CHUNK), jnp.int32),
            pltpu.VMEM((_CHUNK, 8), jnp.float32),
            pltpu.VMEM((_ROWS_PER_TILE // 5, 8), jnp.float32),
            pltpu.VMEM_SHARED((_N_PAD, 8), jnp.float32),
        ],
        compiler_params=_SC_PARAMS,
    )
    ones = jnp.ones((_CHUNK, 8), jnp.float32)
    zeros = jnp.zeros((_ROWS_PER_TILE // 5, 8), jnp.float32)
    return k(dst2d, ones, zeros)


def _pair_readout_call(tab0_d, tab0_a, tab1_d, tab1_a, idx0_2d, idx1_2d, lin_d, lin_a):
    P = idx0_2d.shape[1]

    def body(t0d_h, t0a_h, t1d_h, t1a_h, i0_h, i1_h, ld_h, la_h,
             zd_h, za_h, t0d, t0a, t1d, t1a, i0, i1, ldv, lav, zdv, zav):
        cid = lax.axis_index("c")
        sid = lax.axis_index("s")
        wid = cid * _NS + sid
        pltpu.sync_copy(t0d_h, t0d)
        pltpu.sync_copy(t0a_h, t0a)
        pltpu.sync_copy(t1d_h, t1d)
        pltpu.sync_copy(t1a_h, t1a)
        pltpu.sync_copy(i0_h.at[wid], i0)
        pltpu.sync_copy(i1_h.at[wid], i1)
        pltpu.sync_copy(ld_h.at[wid], ldv)
        pltpu.sync_copy(la_h.at[wid], lav)

        def step(j, _):
            s = pl.ds(j * 16, 16)
            a0 = i0[s]
            a1 = i1[s]
            g0d = plsc.load_gather(t0d, [a0])
            g1d = plsc.load_gather(t1d, [a1])
            g0a = plsc.load_gather(t0a, [a0])
            g1a = plsc.load_gather(t1a, [a1])
            zdv[s] = g0d + g1d + ldv[s]
            zav[s] = g0a + g1a + lav[s]
            return 0

        lax.fori_loop(0, P // 16, step, 0)
        pltpu.sync_copy(zdv, zd_h.at[wid])
        pltpu.sync_copy(zav, za_h.at[wid])

    k = pl.kernel(
        body,
        out_type=[jax.ShapeDtypeStruct((_NW, P), jnp.float32),
                  jax.ShapeDtypeStruct((_NW, P), jnp.float32)],
        mesh=_mesh(),
        scratch_types=[
            pltpu.VMEM((_N_PAD,), jnp.float32),
            pltpu.VMEM((_N_PAD,), jnp.float32),
            pltpu.VMEM((_N_PAD,), jnp.float32),
            pltpu.VMEM((_N_PAD,), jnp.float32),
            pltpu.VMEM((P,), jnp.int32),
            pltpu.VMEM((P,), jnp.int32),
            pltpu.VMEM((P,), jnp.float32),
            pltpu.VMEM((P,), jnp.float32),
            pltpu.VMEM((P,), jnp.float32),
            pltpu.VMEM((P,), jnp.float32),
        ],
        compiler_params=_SC_PARAMS,
    )
    return k(tab0_d, tab0_a, tab1_d, tab1_a, idx0_2d, idx1_2d, lin_d, lin_a)


def _readout_act_kernel(zd_ref, za_ref, dis_ref, ang_ref):
    zd = zd_ref[...]
    za = za_ref[...]
    dis_ref[...] = jax.nn.softplus(zd)
    ang_ref[...] = 2.0 * jax.nn.sigmoid(za)


def _readout_act(zd, za):
    n = zd.shape[0]
    zd = zd.reshape(n // 1000, 1000)
    za = za.reshape(n // 1000, 1000)
    dis, ang = pl.pallas_call(
        _readout_act_kernel,
        out_shape=[jax.ShapeDtypeStruct(zd.shape, jnp.float32),
                   jax.ShapeDtypeStruct(zd.shape, jnp.float32)],
    )(zd, za)
    return dis.reshape(n), ang.reshape(n)


def _pad_table(t):
    return jnp.pad(t, ((0, _N_PAD - t.shape[0]), (0, 0)))


def _edge_split(idx, e_pad, fill):
    e = idx.shape[0]
    p = jnp.pad(idx.astype(jnp.int32), (0, e_pad - e), constant_values=fill)
    return p.reshape(_NC, _NS, e_pad // (_NW * _CHUNK), _CHUNK)


def _epad(e):
    per = -(-e // (_NW * _CHUNK)) * _CHUNK
    return per * _NW


def kernel(cell_feat, net_feat, pin_feat, pins_edge_index, net_net_edge_index, net_net_pair_matrix, net_cell_pair_matrix, cell_lin_W, cell_lin_b, net_lin_W, net_lin_b, pin_lin_W, pin_lin_b, pins_W, pins_b, fa_W, fa_b, so_W, so_b, cf_node_W, cf_node_b, cf_e1_W, cf_e1_b, cf_e2_W, cf_e2_b, cf_out_W, cf_out_b, net_dis_W, net_dis_b, net_angle_W, net_angle_b, pin_dis_W, pin_dis_b, pin_angle_W, pin_angle_b):
    hc = jnp.tanh(cell_feat @ cell_lin_W + cell_lin_b)
    hn = jnp.tanh(net_feat @ net_lin_W + net_lin_b)
    hp = jnp.tanh(pin_feat @ pin_lin_W + pin_lin_b)

    c_idx, n_idx = pins_edge_index[0], pins_edge_index[1]
    f_src, f_dst = net_net_edge_index[0], net_net_edge_index[1]

    ep_pin = _epad(N_PIN)
    ep_nn = _epad(N_NN)
    c2 = _edge_split(c_idx, ep_pin, N_CELL)
    n2 = _edge_split(n_idx, ep_pin, N_NET)
    fs2 = _edge_split(f_src, ep_nn, N_NET)
    fd2 = _edge_split(f_dst, ep_nn, N_NET)

    def inv_sqrt_count(d2):
        cnt = _count_call(d2)
        deg = cnt[0, :, 0] + cnt[1, :, 0]
        return jnp.where(deg > 0, deg, 1.0) ** -0.5

    c_cell_out = inv_sqrt_count(c2)[:N_CELL]
    c_net_in = inv_sqrt_count(n2)[:N_NET]
    c_fsrc = inv_sqrt_count(fs2)[:N_NET]
    c_fdst = inv_sqrt_count(fd2)[:N_NET]

    hp_pad = jnp.pad(hp, ((0, ep_pin - N_PIN), (0, 0)))

    for l in range(L):
        h_pins = _pad_table((hc * c_cell_out[:, None]) @ pins_W[l])
        h_fa = _pad_table((hn * c_fsrc[:, None]) @ fa_W[l])
        h_so = _pad_table((hn * c_fdst[:, None]) @ so_W[l])

        agg = _seg_sum_call(h_pins, c2, n2)
        net_pins = (agg[0, :N_NET] + agg[1, :N_NET]) * c_net_in[:, None] + pins_b[l]
        agg = _seg_sum_call(h_fa, fs2, fd2)
        net_fa = (agg[0, :N_NET] + agg[1, :N_NET]) * c_fdst[:, None] + fa_b[l]
        agg = _seg_sum_call(h_so, fd2, fs2)
        net_so = (agg[0, :N_NET] + agg[1, :N_NET]) * c_fsrc[:, None] + so_b[l]
        new_net = jnp.maximum(jnp.maximum(net_pins, net_fa), net_so)

        hv = _pad_table(hn @ cf_node_W[l] + cf_node_b[l])
        he = _ssp(_ssp(hp_pad @ cf_e1_W[l] + cf_e1_b[l]) @ cf_e2_W[l] + cf_e2_b[l])
        agg = _seg_sum_call(hv, n2, c2, he=he)
        h = agg[0, :N_CELL] + agg[1, :N_CELL]
        new_cell = _ssp(h @ cf_out_W[l] + cf_out_b[l])

        hc, hn = new_cell, new_net

    p0 = net_net_pair_matrix[:, 0].astype(jnp.int32)
    p1 = net_net_pair_matrix[:, 1].astype(jnp.int32)
    q0 = net_cell_pair_matrix[:, 0].astype(jnp.int32)
    q1 = net_cell_pair_matrix[:, 1].astype(jnp.int32)

    hn_pad = _pad_table(hn)
    hc_pad = _pad_table(hc)

    w_nn = jnp.concatenate([net_dis_W[:64], net_angle_W[:64], net_dis_W[64:], net_angle_W[64:]], axis=1)
    s_nn = hn_pad @ w_nn
    t_nn_d0 = s_nn[:, 0] + net_dis_b[0]
    t_nn_a0 = s_nn[:, 1] + net_angle_b[0]
    t_nn_d1 = s_nn[:, 2]
    t_nn_a1 = s_nn[:, 3]

    pnn = ep_nn // _NW
    p0_2d = jnp.pad(p0, (0, ep_nn - N_NN)).reshape(_NW, pnn)
    p1_2d = jnp.pad(p1, (0, ep_nn - N_NN)).reshape(_NW, pnn)
    zeros_nn = jnp.zeros((_NW, pnn), jnp.float32)
    z_nd, z_na = _pair_readout_call(t_nn_d0, t_nn_a0, t_nn_d1, t_nn_a1,
                                    p0_2d, p1_2d, zeros_nn, zeros_nn)
    z_nd = z_nd.reshape(-1)[:N_NN]
    z_na = z_na.reshape(-1)[:N_NN]

    w_np = jnp.concatenate([pin_dis_W[:64], pin_angle_W[:64]], axis=1)
    w_cp = jnp.concatenate([pin_dis_W[80:], pin_angle_W[80:]], axis=1)
    w_pp = jnp.concatenate([pin_dis_W[64:80], pin_angle_W[64:80]], axis=1)
    s_np = hn_pad @ w_np
    s_cp = hc_pad @ w_cp
    s_pp = hp @ w_pp

    ppin = ep_pin // _NW
    q0_2d = jnp.pad(q0, (0, ep_pin - N_PIN)).reshape(_NW, ppin)
    q1_2d = jnp.pad(q1, (0, ep_pin - N_PIN)).reshape(_NW, ppin)
    lin_d = jnp.pad(s_pp[:, 0] + pin_dis_b[0], (0, ep_pin - N_PIN)).reshape(_NW, ppin)
    lin_a = jnp.pad(s_pp[:, 1] + pin_angle_b[0], (0, ep_pin - N_PIN)).reshape(_NW, ppin)
    z_pd, z_pa = _pair_readout_call(s_np[:, 0], s_np[:, 1], s_cp[:, 0], s_cp[:, 1],
                                    q0_2d, q1_2d, lin_d, lin_a)
    z_pd = z_pd.reshape(-1)[:N_PIN]
    z_pa = z_pa.reshape(-1)[:N_PIN]

    net_dis, net_angle = _readout_act(z_nd, z_na)
    pin_dis, pin_angle = _readout_act(z_pd, z_pa)
    return (net_dis, net_angle, pin_dis, pin_angle)
